# baseline (speedup 1.0000x reference)
import functools

import jax
import jax.numpy as jnp
from jax import lax
from jax.experimental import pallas as pl
from jax.experimental.pallas import tpu as pltpu
from jax.experimental.pallas import tpu_sc as plsc

N = 32 * 512 * 512
ROWS, COLS = 16384, 512
BLK_ROWS = 1024
M = 1024
NC, NS, LANES = 2, 16, 16
NW = NC * NS
TILE_ROWS = ROWS // NW
CHUNK_ROWS = 32
NCHUNK = TILE_ROWS // CHUNK_ROWS
VECS = CHUNK_ROWS * COLS // LANES


def _stats_body(x_ref, y_ref, pk_ref, bce_ref, pos_ref, emax_ref):
    i = pl.program_id(0)
    x = x_ref[...]
    y = y_ref[...]
    softplus_negx = jnp.maximum(-x, 0.0) + jnp.log(1.0 + jnp.exp(-jnp.abs(x)))
    bce_blk = jnp.sum(softplus_negx + (1.0 - y) * x)
    pos_blk = jnp.sum(y)
    e = 1.0 - x * (2.0 * y - 1.0)
    emax_blk = jnp.max(e)
    ebits = lax.bitcast_convert_type(e, jnp.uint32)
    packed = (ebits & jnp.uint32(0xFFFFFFFE)) | y.astype(jnp.uint32)
    pk_ref[...] = lax.bitcast_convert_type(packed, jnp.float32)

    @pl.when(i == 0)
    def _():
        bce_ref[0, 0] = bce_blk
        pos_ref[0, 0] = pos_blk
        emax_ref[0, 0] = emax_blk

    @pl.when(i != 0)
    def _():
        bce_ref[0, 0] += bce_blk
        pos_ref[0, 0] += pos_blk
        emax_ref[0, 0] = jnp.maximum(emax_ref[0, 0], emax_blk)


def _hist_body(err_hbm, invw_hbm, out_hbm, ebuf, tbl, ivw, esem):
    wid = lax.axis_index("s") * NC + lax.axis_index("c")
    base = wid * TILE_ROWS
    pltpu.sync_copy(invw_hbm, ivw)

    zeros16 = jnp.zeros((LANES,), jnp.float32)

    def _zrow(r, carry):
        tbl[pl.ds(r * LANES, LANES)] = zeros16
        return carry

    lax.fori_loop(0, 4 * M, _zrow, 0)

    lane = lax.iota(jnp.int32, LANES)
    ones = jnp.ones((LANES,), jnp.float32)
    invw = ivw[...]

    def _start(ci, buf):
        row0 = base + ci * CHUNK_ROWS
        pltpu.make_async_copy(
            err_hbm.at[pl.ds(row0, CHUNK_ROWS), :], ebuf.at[buf],
            esem.at[buf]).start()

    def _wait(ci, buf):
        row0 = base + ci * CHUNK_ROWS
        pltpu.make_async_copy(
            err_hbm.at[pl.ds(row0, CHUNK_ROWS), :], ebuf.at[buf],
            esem.at[buf]).wait()

    _start(0, 0)

    def _chunk(ci, carry):
        cur = lax.bitwise_and(ci, 1)
        _wait(ci, cur)

        @pl.when(ci + 1 < NCHUNK)
        def _():
            _start(ci + 1, 1 - cur)

        def _vec(vi, c2):
            r = lax.shift_right_logical(vi, 5)
            c = lax.shift_left(lax.bitwise_and(vi, 31), 4)
            raw = ebuf[cur, r, pl.ds(c, LANES)]
            bits = lax.bitcast_convert_type(raw, jnp.uint32)
            cls = (bits & jnp.uint32(1)).astype(jnp.int32)
            ev = lax.bitcast_convert_type(
                bits & jnp.uint32(0xFFFFFFFE), jnp.float32)
            tpos = ev * invw
            j = jnp.clip(tpos.astype(jnp.int32), 0, M - 1)
            frac = tpos - j.astype(jnp.float32)
            mall = ev > 0.0
            idx = j * LANES + lane + cls * (2 * M * LANES)
            plsc.addupdate_scatter(tbl, [idx], ones, mask=mall)
            plsc.addupdate_scatter(tbl, [idx + (M * LANES)], frac, mask=mall)
            return c2

        lax.fori_loop(0, VECS, _vec, 0, unroll=8)
        return carry

    lax.fori_loop(0, NCHUNK, _chunk, 0)
    pltpu.sync_copy(tbl, out_hbm.at[pl.ds(wid * (4 * M * LANES), 4 * M * LANES)])


def _combine_body(h_ref, bce_ref, pos_ref, emax_ref, out_ref):
    h = h_ref[...]
    hs = jnp.sum(h, axis=1)
    c = hs[0:1, :]
    s = hs[1:2, :]
    m = c + hs[2:3, :]
    S = s + hs[3:4, :]
    P = pos_ref[0, 0]
    emax = emax_ref[0, 0]
    w = jnp.maximum(emax, 1e-30) * (1.0 / M)

    row = lax.broadcasted_iota(jnp.int32, (M, M), 0)
    col = lax.broadcasted_iota(jnp.int32, (M, M), 1)
    V0 = (row >= col).astype(jnp.float32)
    V1 = (row > col).astype(jnp.float32)
    VL = (row < col).astype(jnp.float32)
    dot = functools.partial(lax.dot, precision=lax.Precision.HIGHEST)

    D0 = P + dot(c, V0)
    D1 = P + dot(c, V1)
    ybar = s / jnp.maximum(c, 1.0)
    dF = w * ((1.0 - ybar) / jnp.maximum(D1, 1.0) + ybar / jnp.maximum(D0, 1.0))
    F = dot(dF, VL)
    lov = jnp.sum(m * F + S * dF)
    out_ref[0, 0] = bce_ref[0, 0] * (1.0 / N) + lov


def kernel(logits, targets):
    x2 = logits.reshape(ROWS, COLS)
    y2 = targets.reshape(ROWS, COLS)

    packed, bce, pos, emax = pl.pallas_call(
        _stats_body,
        grid=(ROWS // BLK_ROWS,),
        in_specs=[
            pl.BlockSpec((BLK_ROWS, COLS), lambda i: (i, 0)),
            pl.BlockSpec((BLK_ROWS, COLS), lambda i: (i, 0)),
        ],
        out_specs=[
            pl.BlockSpec((BLK_ROWS, COLS), lambda i: (i, 0)),
            pl.BlockSpec((1, 1), lambda i: (0, 0), memory_space=pltpu.SMEM),
            pl.BlockSpec((1, 1), lambda i: (0, 0), memory_space=pltpu.SMEM),
            pl.BlockSpec((1, 1), lambda i: (0, 0), memory_space=pltpu.SMEM),
        ],
        out_shape=[
            jax.ShapeDtypeStruct((ROWS, COLS), jnp.float32),
            jax.ShapeDtypeStruct((1, 1), jnp.float32),
            jax.ShapeDtypeStruct((1, 1), jnp.float32),
            jax.ShapeDtypeStruct((1, 1), jnp.float32),
        ],
        compiler_params=pltpu.CompilerParams(
            dimension_semantics=("arbitrary",)),
    )(x2, y2)

    invw = jnp.float32(M) / jnp.maximum(emax[0, 0], jnp.float32(1e-30))
    invw_vec = jnp.full((LANES,), invw, jnp.float32)

    hist = pl.kernel(
        _hist_body,
        out_type=jax.ShapeDtypeStruct((NW * 4 * M * LANES,), jnp.float32),
        mesh=plsc.VectorSubcoreMesh(core_axis_name="c", subcore_axis_name="s"),
        scratch_types=[
            pltpu.VMEM((2, CHUNK_ROWS, COLS), jnp.float32),
            pltpu.VMEM((4 * M * LANES,), jnp.float32),
            pltpu.VMEM((LANES,), jnp.float32),
            pltpu.SemaphoreType.DMA((2,)),
        ],
        compiler_params=pltpu.CompilerParams(
            needs_layout_passes=False, use_tc_tiling_on_sc=True),
    )(packed, invw_vec)
    hist = hist.reshape(NW, 4, M, LANES).transpose(1, 0, 3, 2)
    hist = hist.reshape(4, NW * LANES, M)

    out = pl.pallas_call(
        _combine_body,
        in_specs=[
            pl.BlockSpec(memory_space=pltpu.VMEM),
            pl.BlockSpec(memory_space=pltpu.SMEM),
            pl.BlockSpec(memory_space=pltpu.SMEM),
            pl.BlockSpec(memory_space=pltpu.SMEM),
        ],
        out_specs=pl.BlockSpec(memory_space=pltpu.SMEM),
        out_shape=jax.ShapeDtypeStruct((1, 1), jnp.float32),
    )(hist, bce, pos, emax)
    return out[0, 0]


# speedup vs baseline: 1.0086x; 1.0086x over previous
import functools

import jax
import jax.numpy as jnp
from jax import lax
from jax.experimental import pallas as pl
from jax.experimental.pallas import tpu as pltpu
from jax.experimental.pallas import tpu_sc as plsc

N = 32 * 512 * 512
ROWS, COLS = 16384, 512
BLK_ROWS = 1024
M = 1024
NC, NS, LANES = 2, 16, 16
NW = NC * NS
TILE_ROWS = ROWS // NW
CHUNK_ROWS = 32
NCHUNK = TILE_ROWS // CHUNK_ROWS
VECS = CHUNK_ROWS * COLS // LANES


def _stats_body(x_ref, y_ref, pk_ref, bce_ref, pos_ref, emax_ref):
    i = pl.program_id(0)
    x = x_ref[...]
    y = y_ref[...]
    softplus_negx = jnp.maximum(-x, 0.0) + jnp.log(1.0 + jnp.exp(-jnp.abs(x)))
    bce_blk = jnp.sum(softplus_negx + (1.0 - y) * x)
    pos_blk = jnp.sum(y)
    e = 1.0 - x * (2.0 * y - 1.0)
    emax_blk = jnp.max(e)
    ebits = lax.bitcast_convert_type(e, jnp.uint32)
    packed = (ebits & jnp.uint32(0xFFFFFFFE)) | y.astype(jnp.uint32)
    pk_ref[...] = lax.bitcast_convert_type(packed, jnp.float32)

    @pl.when(i == 0)
    def _():
        bce_ref[0, 0] = bce_blk
        pos_ref[0, 0] = pos_blk
        emax_ref[0, 0] = emax_blk

    @pl.when(i != 0)
    def _():
        bce_ref[0, 0] += bce_blk
        pos_ref[0, 0] += pos_blk
        emax_ref[0, 0] = jnp.maximum(emax_ref[0, 0], emax_blk)


def _hist_body(err_hbm, invw_hbm, out_hbm, ebuf, tbl, ivw, esem):
    wid = lax.axis_index("s") * NC + lax.axis_index("c")
    base = wid * TILE_ROWS
    pltpu.sync_copy(invw_hbm, ivw)

    zeros16 = jnp.zeros((LANES,), jnp.float32)

    def _zrow(r, carry):
        tbl[pl.ds(r * LANES, LANES)] = zeros16
        return carry

    lax.fori_loop(0, 4 * M, _zrow, 0)

    lane = lax.iota(jnp.int32, LANES)
    ones = jnp.ones((LANES,), jnp.float32)
    invw = ivw[...]

    def _start(ci, buf):
        row0 = base + ci * CHUNK_ROWS
        pltpu.make_async_copy(
            err_hbm.at[pl.ds(row0, CHUNK_ROWS), :], ebuf.at[buf],
            esem.at[buf]).start()

    def _wait(ci, buf):
        row0 = base + ci * CHUNK_ROWS
        pltpu.make_async_copy(
            err_hbm.at[pl.ds(row0, CHUNK_ROWS), :], ebuf.at[buf],
            esem.at[buf]).wait()

    _start(0, 0)

    def _chunk(ci, carry):
        cur = lax.bitwise_and(ci, 1)
        _wait(ci, cur)

        @pl.when(ci + 1 < NCHUNK)
        def _():
            _start(ci + 1, 1 - cur)

        def _row(r, c2):
            for cg in range(COLS // LANES):
                raw = ebuf[cur, r, pl.ds(cg * LANES, LANES)]
                bits = lax.bitcast_convert_type(raw, jnp.uint32)
                cls = (bits & jnp.uint32(1)).astype(jnp.int32)
                ev = lax.bitcast_convert_type(
                    bits & jnp.uint32(0xFFFFFFFE), jnp.float32)
                tpos = ev * invw
                j = jnp.clip(tpos.astype(jnp.int32), 0, M - 1)
                frac = tpos - j.astype(jnp.float32)
                mall = ev > 0.0
                idx = j * LANES + lane + cls * (2 * M * LANES)
                plsc.addupdate_scatter(tbl, [idx], ones, mask=mall)
                plsc.addupdate_scatter(tbl, [idx + (M * LANES)], frac, mask=mall)
            return c2

        lax.fori_loop(0, CHUNK_ROWS, _row, 0)
        return carry

    lax.fori_loop(0, NCHUNK, _chunk, 0)
    pltpu.sync_copy(tbl, out_hbm.at[pl.ds(wid * (4 * M * LANES), 4 * M * LANES)])


def _combine_body(h_ref, bce_ref, pos_ref, emax_ref, out_ref):
    h = h_ref[...]
    hs = jnp.sum(h, axis=1)
    c = hs[0:1, :]
    s = hs[1:2, :]
    m = c + hs[2:3, :]
    S = s + hs[3:4, :]
    P = pos_ref[0, 0]
    emax = emax_ref[0, 0]
    w = jnp.maximum(emax, 1e-30) * (1.0 / M)

    row = lax.broadcasted_iota(jnp.int32, (M, M), 0)
    col = lax.broadcasted_iota(jnp.int32, (M, M), 1)
    V0 = (row >= col).astype(jnp.float32)
    V1 = (row > col).astype(jnp.float32)
    VL = (row < col).astype(jnp.float32)
    dot = functools.partial(lax.dot, precision=lax.Precision.HIGHEST)

    D0 = P + dot(c, V0)
    D1 = P + dot(c, V1)
    ybar = s / jnp.maximum(c, 1.0)
    dF = w * ((1.0 - ybar) / jnp.maximum(D1, 1.0) + ybar / jnp.maximum(D0, 1.0))
    F = dot(dF, VL)
    lov = jnp.sum(m * F + S * dF)
    out_ref[0, 0] = bce_ref[0, 0] * (1.0 / N) + lov


def kernel(logits, targets):
    x2 = logits.reshape(ROWS, COLS)
    y2 = targets.reshape(ROWS, COLS)

    packed, bce, pos, emax = pl.pallas_call(
        _stats_body,
        grid=(ROWS // BLK_ROWS,),
        in_specs=[
            pl.BlockSpec((BLK_ROWS, COLS), lambda i: (i, 0)),
            pl.BlockSpec((BLK_ROWS, COLS), lambda i: (i, 0)),
        ],
        out_specs=[
            pl.BlockSpec((BLK_ROWS, COLS), lambda i: (i, 0)),
            pl.BlockSpec((1, 1), lambda i: (0, 0), memory_space=pltpu.SMEM),
            pl.BlockSpec((1, 1), lambda i: (0, 0), memory_space=pltpu.SMEM),
            pl.BlockSpec((1, 1), lambda i: (0, 0), memory_space=pltpu.SMEM),
        ],
        out_shape=[
            jax.ShapeDtypeStruct((ROWS, COLS), jnp.float32),
            jax.ShapeDtypeStruct((1, 1), jnp.float32),
            jax.ShapeDtypeStruct((1, 1), jnp.float32),
            jax.ShapeDtypeStruct((1, 1), jnp.float32),
        ],
        compiler_params=pltpu.CompilerParams(
            dimension_semantics=("arbitrary",)),
    )(x2, y2)

    invw = jnp.float32(M) / jnp.maximum(emax[0, 0], jnp.float32(1e-30))
    invw_vec = jnp.full((LANES,), invw, jnp.float32)

    hist = pl.kernel(
        _hist_body,
        out_type=jax.ShapeDtypeStruct((NW * 4 * M * LANES,), jnp.float32),
        mesh=plsc.VectorSubcoreMesh(core_axis_name="c", subcore_axis_name="s"),
        scratch_types=[
            pltpu.VMEM((2, CHUNK_ROWS, COLS), jnp.float32),
            pltpu.VMEM((4 * M * LANES,), jnp.float32),
            pltpu.VMEM((LANES,), jnp.float32),
            pltpu.SemaphoreType.DMA((2,)),
        ],
        compiler_params=pltpu.CompilerParams(
            needs_layout_passes=False, use_tc_tiling_on_sc=True),
    )(packed, invw_vec)
    hist = hist.reshape(NW, 4, M, LANES).transpose(1, 0, 3, 2)
    hist = hist.reshape(4, NW * LANES, M)

    out = pl.pallas_call(
        _combine_body,
        in_specs=[
            pl.BlockSpec(memory_space=pltpu.VMEM),
            pl.BlockSpec(memory_space=pltpu.SMEM),
            pl.BlockSpec(memory_space=pltpu.SMEM),
            pl.BlockSpec(memory_space=pltpu.SMEM),
        ],
        out_specs=pl.BlockSpec(memory_space=pltpu.SMEM),
        out_shape=jax.ShapeDtypeStruct((1, 1), jnp.float32),
    )(hist, bce, pos, emax)
    return out[0, 0]


# speedup vs baseline: 1.4176x; 1.4056x over previous
import functools

import jax
import jax.numpy as jnp
from jax import lax
from jax.experimental import pallas as pl
from jax.experimental.pallas import tpu as pltpu
from jax.experimental.pallas import tpu_sc as plsc

N = 32 * 512 * 512
ROWS, COLS = 16384, 512
BLK_ROWS = 1024
M = 1024
NC, NS, LANES = 2, 16, 16
NW = NC * NS
TILE_ROWS = ROWS // NW
CHUNK_ROWS = 32
NCHUNK = TILE_ROWS // CHUNK_ROWS
VECS = CHUNK_ROWS * COLS // LANES


def _stats_body(x_ref, y_ref, pk_ref, bce_ref, pos_ref, emax_ref):
    i = pl.program_id(0)
    x = x_ref[...]
    y = y_ref[...]
    softplus_negx = jnp.maximum(-x, 0.0) + jnp.log(1.0 + jnp.exp(-jnp.abs(x)))
    bce_blk = jnp.sum(softplus_negx + (1.0 - y) * x)
    pos_blk = jnp.sum(y)
    e = 1.0 - x * (2.0 * y - 1.0)
    emax_blk = jnp.max(e)
    ebits = lax.bitcast_convert_type(e, jnp.uint32)
    packed = (ebits & jnp.uint32(0xFFFFFFFE)) | y.astype(jnp.uint32)
    pk_ref[...] = lax.bitcast_convert_type(packed, jnp.float32)

    @pl.when(i == 0)
    def _():
        bce_ref[0, 0] = bce_blk
        pos_ref[0, 0] = pos_blk
        emax_ref[0, 0] = emax_blk

    @pl.when(i != 0)
    def _():
        bce_ref[0, 0] += bce_blk
        pos_ref[0, 0] += pos_blk
        emax_ref[0, 0] = jnp.maximum(emax_ref[0, 0], emax_blk)


def _hist_body(err_hbm, invw_hbm, out_hbm, ebuf, tbl, ivw, esem):
    wid = lax.axis_index("s") * NC + lax.axis_index("c")
    base = wid * TILE_ROWS
    pltpu.sync_copy(invw_hbm, ivw)

    zeros16 = jnp.zeros((LANES,), jnp.float32)

    def _zrow(r, carry):
        tbl[pl.ds(r * LANES, LANES)] = zeros16
        return carry

    lax.fori_loop(0, 4 * M, _zrow, 0)

    lane = lax.iota(jnp.int32, LANES)
    ones = jnp.ones((LANES,), jnp.float32)
    invw = ivw[...]

    def _start(ci, buf):
        row0 = base + ci * CHUNK_ROWS
        pltpu.make_async_copy(
            err_hbm.at[pl.ds(row0, CHUNK_ROWS), :], ebuf.at[buf],
            esem.at[buf]).start()

    def _wait(ci, buf):
        row0 = base + ci * CHUNK_ROWS
        pltpu.make_async_copy(
            err_hbm.at[pl.ds(row0, CHUNK_ROWS), :], ebuf.at[buf],
            esem.at[buf]).wait()

    _start(0, 0)

    def _chunk(ci, carry):
        cur = lax.bitwise_and(ci, 1)
        _wait(ci, cur)

        @pl.when(ci + 1 < NCHUNK)
        def _():
            _start(ci + 1, 1 - cur)

        @plsc.parallel_loop(0, CHUNK_ROWS, unroll=2)
        def _row(r):
            for cg in range(COLS // LANES):
                raw = ebuf[cur, r, pl.ds(cg * LANES, LANES)]
                bits = lax.bitcast_convert_type(raw, jnp.uint32)
                cls = (bits & jnp.uint32(1)).astype(jnp.int32)
                ev = lax.bitcast_convert_type(
                    bits & jnp.uint32(0xFFFFFFFE), jnp.float32)
                tpos = ev * invw
                j = jnp.clip(tpos.astype(jnp.int32), 0, M - 1)
                frac = tpos - j.astype(jnp.float32)
                mall = ev > 0.0
                idx = j * LANES + lane + cls * (2 * M * LANES)
                plsc.addupdate_scatter(tbl, [idx], ones, mask=mall)
                plsc.addupdate_scatter(tbl, [idx + (M * LANES)], frac, mask=mall)

        return carry

    lax.fori_loop(0, NCHUNK, _chunk, 0)
    pltpu.sync_copy(tbl, out_hbm.at[pl.ds(wid * (4 * M * LANES), 4 * M * LANES)])


def _combine_body(h_ref, bce_ref, pos_ref, emax_ref, out_ref):
    h = h_ref[...]
    hs = jnp.sum(h, axis=1)
    c = hs[0:1, :]
    s = hs[1:2, :]
    m = c + hs[2:3, :]
    S = s + hs[3:4, :]
    P = pos_ref[0, 0]
    emax = emax_ref[0, 0]
    w = jnp.maximum(emax, 1e-30) * (1.0 / M)

    row = lax.broadcasted_iota(jnp.int32, (M, M), 0)
    col = lax.broadcasted_iota(jnp.int32, (M, M), 1)
    V0 = (row >= col).astype(jnp.float32)
    V1 = (row > col).astype(jnp.float32)
    VL = (row < col).astype(jnp.float32)
    dot = functools.partial(lax.dot, precision=lax.Precision.HIGHEST)

    D0 = P + dot(c, V0)
    D1 = P + dot(c, V1)
    ybar = s / jnp.maximum(c, 1.0)
    dF = w * ((1.0 - ybar) / jnp.maximum(D1, 1.0) + ybar / jnp.maximum(D0, 1.0))
    F = dot(dF, VL)
    lov = jnp.sum(m * F + S * dF)
    out_ref[0, 0] = bce_ref[0, 0] * (1.0 / N) + lov


def kernel(logits, targets):
    x2 = logits.reshape(ROWS, COLS)
    y2 = targets.reshape(ROWS, COLS)

    packed, bce, pos, emax = pl.pallas_call(
        _stats_body,
        grid=(ROWS // BLK_ROWS,),
        in_specs=[
            pl.BlockSpec((BLK_ROWS, COLS), lambda i: (i, 0)),
            pl.BlockSpec((BLK_ROWS, COLS), lambda i: (i, 0)),
        ],
        out_specs=[
            pl.BlockSpec((BLK_ROWS, COLS), lambda i: (i, 0)),
            pl.BlockSpec((1, 1), lambda i: (0, 0), memory_space=pltpu.SMEM),
            pl.BlockSpec((1, 1), lambda i: (0, 0), memory_space=pltpu.SMEM),
            pl.BlockSpec((1, 1), lambda i: (0, 0), memory_space=pltpu.SMEM),
        ],
        out_shape=[
            jax.ShapeDtypeStruct((ROWS, COLS), jnp.float32),
            jax.ShapeDtypeStruct((1, 1), jnp.float32),
            jax.ShapeDtypeStruct((1, 1), jnp.float32),
            jax.ShapeDtypeStruct((1, 1), jnp.float32),
        ],
        compiler_params=pltpu.CompilerParams(
            dimension_semantics=("arbitrary",)),
    )(x2, y2)

    invw = jnp.float32(M) / jnp.maximum(emax[0, 0], jnp.float32(1e-30))
    invw_vec = jnp.full((LANES,), invw, jnp.float32)

    hist = pl.kernel(
        _hist_body,
        out_type=jax.ShapeDtypeStruct((NW * 4 * M * LANES,), jnp.float32),
        mesh=plsc.VectorSubcoreMesh(core_axis_name="c", subcore_axis_name="s"),
        scratch_types=[
            pltpu.VMEM((2, CHUNK_ROWS, COLS), jnp.float32),
            pltpu.VMEM((4 * M * LANES,), jnp.float32),
            pltpu.VMEM((LANES,), jnp.float32),
            pltpu.SemaphoreType.DMA((2,)),
        ],
        compiler_params=pltpu.CompilerParams(
            needs_layout_passes=False, use_tc_tiling_on_sc=True),
    )(packed, invw_vec)
    hist = hist.reshape(NW, 4, M, LANES).transpose(1, 0, 3, 2)
    hist = hist.reshape(4, NW * LANES, M)

    out = pl.pallas_call(
        _combine_body,
        in_specs=[
            pl.BlockSpec(memory_space=pltpu.VMEM),
            pl.BlockSpec(memory_space=pltpu.SMEM),
            pl.BlockSpec(memory_space=pltpu.SMEM),
            pl.BlockSpec(memory_space=pltpu.SMEM),
        ],
        out_specs=pl.BlockSpec(memory_space=pltpu.SMEM),
        out_shape=jax.ShapeDtypeStruct((1, 1), jnp.float32),
    )(hist, bce, pos, emax)
    return out[0, 0]


# speedup vs baseline: 1.6340x; 1.1527x over previous
import functools

import jax
import jax.numpy as jnp
from jax import lax
from jax.experimental import pallas as pl
from jax.experimental.pallas import tpu as pltpu
from jax.experimental.pallas import tpu_sc as plsc

N = 32 * 512 * 512
ROWS, COLS = 16384, 512
BLK_ROWS = 1024
M = 1024
NC, NS, LANES = 2, 16, 16
NW = NC * NS
TILE_ROWS = ROWS // NW
CHUNK_ROWS = 32
NCHUNK = TILE_ROWS // CHUNK_ROWS
VECS = CHUNK_ROWS * COLS // LANES


def _stats_body(x_ref, y_ref, pk_ref, bce_ref, pos_ref, emax_ref):
    i = pl.program_id(0)
    x = x_ref[...]
    y = y_ref[...]
    softplus_negx = jnp.maximum(-x, 0.0) + jnp.log(1.0 + jnp.exp(-jnp.abs(x)))
    bce_blk = jnp.sum(softplus_negx + (1.0 - y) * x)
    pos_blk = jnp.sum(y)
    e = 1.0 - x * (2.0 * y - 1.0)
    emax_blk = jnp.max(e)
    ebits = lax.bitcast_convert_type(e, jnp.uint32)
    packed = (ebits & jnp.uint32(0xFFFFFFFE)) | y.astype(jnp.uint32)
    pk_ref[...] = lax.bitcast_convert_type(packed, jnp.float32)

    @pl.when(i == 0)
    def _():
        bce_ref[0, 0] = bce_blk
        pos_ref[0, 0] = pos_blk
        emax_ref[0, 0] = emax_blk

    @pl.when(i != 0)
    def _():
        bce_ref[0, 0] += bce_blk
        pos_ref[0, 0] += pos_blk
        emax_ref[0, 0] = jnp.maximum(emax_ref[0, 0], emax_blk)


def _hist_body(err_hbm, invw_hbm, out_hbm, ebuf, tbl, ivw, esem):
    wid = lax.axis_index("s") * NC + lax.axis_index("c")
    base = wid * TILE_ROWS
    pltpu.sync_copy(invw_hbm, ivw)

    zeros16 = jnp.zeros((LANES,), jnp.float32)

    def _zrow(r, carry):
        tbl[pl.ds(r * LANES, LANES)] = zeros16
        return carry

    lax.fori_loop(0, 4 * M, _zrow, 0)

    lane = lax.iota(jnp.int32, LANES)
    ones = jnp.ones((LANES,), jnp.float32)
    invw = ivw[...]

    def _start(ci, buf):
        row0 = base + ci * CHUNK_ROWS
        pltpu.make_async_copy(
            err_hbm.at[pl.ds(row0, CHUNK_ROWS), :], ebuf.at[buf],
            esem.at[buf]).start()

    def _wait(ci, buf):
        row0 = base + ci * CHUNK_ROWS
        pltpu.make_async_copy(
            err_hbm.at[pl.ds(row0, CHUNK_ROWS), :], ebuf.at[buf],
            esem.at[buf]).wait()

    _start(0, 0)

    def _chunk(ci, carry):
        cur = lax.bitwise_and(ci, 1)
        _wait(ci, cur)

        @pl.when(ci + 1 < NCHUNK)
        def _():
            _start(ci + 1, 1 - cur)

        @plsc.parallel_loop(0, CHUNK_ROWS, unroll=4)
        def _row(r):
            for cg in range(COLS // LANES):
                raw = ebuf[cur, r, pl.ds(cg * LANES, LANES)]
                bits = lax.bitcast_convert_type(raw, jnp.uint32)
                cls = (bits & jnp.uint32(1)).astype(jnp.int32)
                ev = lax.bitcast_convert_type(
                    bits & jnp.uint32(0xFFFFFFFE), jnp.float32)
                tpos = ev * invw
                j = jnp.clip(tpos.astype(jnp.int32), 0, M - 1)
                frac = tpos - j.astype(jnp.float32)
                mall = ev > 0.0
                idx = j * LANES + lane + cls * (2 * M * LANES)
                plsc.addupdate_scatter(tbl, [idx], ones, mask=mall)
                plsc.addupdate_scatter(tbl, [idx + (M * LANES)], frac, mask=mall)

        return carry

    lax.fori_loop(0, NCHUNK, _chunk, 0)
    pltpu.sync_copy(tbl, out_hbm.at[pl.ds(wid * (4 * M * LANES), 4 * M * LANES)])


def _combine_body(h_ref, bce_ref, pos_ref, emax_ref, out_ref):
    h = h_ref[...]
    hs = jnp.sum(h, axis=1)
    c = hs[0:1, :]
    s = hs[1:2, :]
    m = c + hs[2:3, :]
    S = s + hs[3:4, :]
    P = pos_ref[0, 0]
    emax = emax_ref[0, 0]
    w = jnp.maximum(emax, 1e-30) * (1.0 / M)

    row = lax.broadcasted_iota(jnp.int32, (M, M), 0)
    col = lax.broadcasted_iota(jnp.int32, (M, M), 1)
    V0 = (row >= col).astype(jnp.float32)
    V1 = (row > col).astype(jnp.float32)
    VL = (row < col).astype(jnp.float32)
    dot = functools.partial(lax.dot, precision=lax.Precision.HIGHEST)

    D0 = P + dot(c, V0)
    D1 = P + dot(c, V1)
    ybar = s / jnp.maximum(c, 1.0)
    dF = w * ((1.0 - ybar) / jnp.maximum(D1, 1.0) + ybar / jnp.maximum(D0, 1.0))
    F = dot(dF, VL)
    lov = jnp.sum(m * F + S * dF)
    out_ref[0, 0] = bce_ref[0, 0] * (1.0 / N) + lov


def kernel(logits, targets):
    x2 = logits.reshape(ROWS, COLS)
    y2 = targets.reshape(ROWS, COLS)

    packed, bce, pos, emax = pl.pallas_call(
        _stats_body,
        grid=(ROWS // BLK_ROWS,),
        in_specs=[
            pl.BlockSpec((BLK_ROWS, COLS), lambda i: (i, 0)),
            pl.BlockSpec((BLK_ROWS, COLS), lambda i: (i, 0)),
        ],
        out_specs=[
            pl.BlockSpec((BLK_ROWS, COLS), lambda i: (i, 0)),
            pl.BlockSpec((1, 1), lambda i: (0, 0), memory_space=pltpu.SMEM),
            pl.BlockSpec((1, 1), lambda i: (0, 0), memory_space=pltpu.SMEM),
            pl.BlockSpec((1, 1), lambda i: (0, 0), memory_space=pltpu.SMEM),
        ],
        out_shape=[
            jax.ShapeDtypeStruct((ROWS, COLS), jnp.float32),
            jax.ShapeDtypeStruct((1, 1), jnp.float32),
            jax.ShapeDtypeStruct((1, 1), jnp.float32),
            jax.ShapeDtypeStruct((1, 1), jnp.float32),
        ],
        compiler_params=pltpu.CompilerParams(
            dimension_semantics=("arbitrary",)),
    )(x2, y2)

    invw = jnp.float32(M) / jnp.maximum(emax[0, 0], jnp.float32(1e-30))
    invw_vec = jnp.full((LANES,), invw, jnp.float32)

    hist = pl.kernel(
        _hist_body,
        out_type=jax.ShapeDtypeStruct((NW * 4 * M * LANES,), jnp.float32),
        mesh=plsc.VectorSubcoreMesh(core_axis_name="c", subcore_axis_name="s"),
        scratch_types=[
            pltpu.VMEM((2, CHUNK_ROWS, COLS), jnp.float32),
            pltpu.VMEM((4 * M * LANES,), jnp.float32),
            pltpu.VMEM((LANES,), jnp.float32),
            pltpu.SemaphoreType.DMA((2,)),
        ],
        compiler_params=pltpu.CompilerParams(
            needs_layout_passes=False, use_tc_tiling_on_sc=True),
    )(packed, invw_vec)
    hist = hist.reshape(NW, 4, M, LANES).transpose(1, 0, 3, 2)
    hist = hist.reshape(4, NW * LANES, M)

    out = pl.pallas_call(
        _combine_body,
        in_specs=[
            pl.BlockSpec(memory_space=pltpu.VMEM),
            pl.BlockSpec(memory_space=pltpu.SMEM),
            pl.BlockSpec(memory_space=pltpu.SMEM),
            pl.BlockSpec(memory_space=pltpu.SMEM),
        ],
        out_specs=pl.BlockSpec(memory_space=pltpu.SMEM),
        out_shape=jax.ShapeDtypeStruct((1, 1), jnp.float32),
    )(hist, bce, pos, emax)
    return out[0, 0]
